# baseline (speedup 1.0000x reference)
import jax
import jax.numpy as jnp
from jax import lax
from jax.experimental import pallas as pl
from jax.experimental.pallas import tpu as pltpu
from jax.experimental.pallas import tpu_sc as plsc

X_MIN, Y_MIN, Z_MIN = 0.0, -50.0, -2.0
GRID_INV = 2.5

B, N, C = 8, 131072, 5
ROWS = B * N
OUT_C = 4

NUM_CORES = 2
NUM_SUBCORES = 16
NW = NUM_CORES * NUM_SUBCORES
ROWS_PER_W = ROWS // NW
CHUNK = 8192
N_CHUNKS = ROWS_PER_W // CHUNK
LANES = 16

_MINS = (X_MIN, Y_MIN, Z_MIN)


def _sc_body(in_hbm, feat_hbm, idx_hbm, *scratch):
    ins = (scratch[0:4], scratch[4:8])
    qs = (scratch[8:11], scratch[11:14])
    b_v = scratch[14]
    sem_in = scratch[15:17]
    sem_feat = scratch[17:19]
    sem_idx = scratch[19:21]
    sem_b = scratch[21]

    cid = lax.axis_index("c")
    sid = lax.axis_index("s")
    wid = sid * NUM_CORES + cid
    row0 = wid * ROWS_PER_W
    b = row0 // N
    n_off = row0 % N

    ones = jnp.ones((LANES,), jnp.int32)
    zeros = jnp.zeros((LANES,), jnp.int32)
    bvec = zeros + b

    @plsc.parallel_loop(0, CHUNK // LANES, unroll=8)
    def fill_b(i):
        b_v[pl.ds(i * LANES, LANES)] = bvec

    b_descs = [
        pltpu.async_copy(b_v, idx_hbm.at[0, pl.ds(row0 + k * CHUNK, CHUNK)],
                         sem_b)
        for k in range(N_CHUNKS)
    ]

    def issue_in(k):
        s = k % 2
        return [
            pltpu.async_copy(
                in_hbm.at[c, b, pl.ds(n_off + k * CHUNK, CHUNK)],
                ins[s][c], sem_in[s])
            for c in range(4)
        ]

    def issue_out(k):
        s = k % 2
        feat = [
            pltpu.async_copy(
                ins[s][c], feat_hbm.at[c, pl.ds(row0 + k * CHUNK, CHUNK)],
                sem_feat[s])
            for c in range(4)
        ]
        idx = [
            pltpu.async_copy(
                qs[s][c], idx_hbm.at[3 - c, pl.ds(row0 + k * CHUNK, CHUNK)],
                sem_idx[s])
            for c in range(3)
        ]
        return feat + idx

    def compute(k):
        s = k % 2

        @plsc.parallel_loop(0, CHUNK // LANES, unroll=16)
        def quant(i):
            sl = pl.ds(i * LANES, LANES)
            for c in range(3):
                v = (ins[s][c][sl] - _MINS[c]) * GRID_INV
                t = lax.convert_element_type(v, jnp.int32)
                tf = lax.convert_element_type(t, jnp.float32)
                qs[s][c][sl] = t + lax.select(v > tf, ones, zeros)

    in_d = {0: issue_in(0)}
    out_d = {}
    for k in range(N_CHUNKS):
        if k + 1 < N_CHUNKS:
            if k >= 1:
                for d in out_d.pop(k - 1):
                    d.wait()
            in_d[k + 1] = issue_in(k + 1)
        for d in in_d.pop(k):
            d.wait()
        compute(k)
        out_d[k] = issue_out(k)
    for key in sorted(out_d):
        for d in out_d[key]:
            d.wait()
    for d in b_descs:
        d.wait()


@jax.jit
def kernel(rdr_sparse_cube):
    planar = jnp.transpose(rdr_sparse_cube, (2, 0, 1))
    mesh = plsc.VectorSubcoreMesh(
        core_axis_name="c", subcore_axis_name="s",
        num_cores=NUM_CORES, num_subcores=NUM_SUBCORES)
    feat_t, idx_t = pl.kernel(
        _sc_body,
        out_type=(
            jax.ShapeDtypeStruct((OUT_C, ROWS), jnp.float32),
            jax.ShapeDtypeStruct((OUT_C, ROWS), jnp.int32),
        ),
        mesh=mesh,
        scratch_types=(
            [pltpu.VMEM((CHUNK,), jnp.float32) for _ in range(8)]
            + [pltpu.VMEM((CHUNK,), jnp.int32) for _ in range(6)]
            + [pltpu.VMEM((CHUNK,), jnp.int32)]
            + [pltpu.SemaphoreType.DMA] * 7
        ),
    )(planar)
    return feat_t.T, idx_t.T


# speedup vs baseline: 1.0556x; 1.0556x over previous
import jax
import jax.numpy as jnp
from jax import lax
from jax.experimental import pallas as pl
from jax.experimental.pallas import tpu as pltpu
from jax.experimental.pallas import tpu_sc as plsc

X_MIN, Y_MIN, Z_MIN = 0.0, -50.0, -2.0
GRID_INV = 2.5

B, N, C = 8, 131072, 5
ROWS = B * N
OUT_C = 4

NUM_CORES = 2
NUM_SUBCORES = 16
NW = NUM_CORES * NUM_SUBCORES
ROWS_PER_W = ROWS // NW
CHUNK = 8192
N_CHUNKS = ROWS_PER_W // CHUNK
LANES = 16

_MINS = (X_MIN, Y_MIN, Z_MIN)


def _sc_body(in_hbm, feat_hbm, idx_hbm, *scratch):
    ins = (scratch[0:4], scratch[4:8])
    qs = (scratch[8:11], scratch[11:14])
    b_v = scratch[14]
    sem_in = scratch[15:17]
    sem_feat = scratch[17:19]
    sem_idx = scratch[19:21]
    sem_b = scratch[21]

    cid = lax.axis_index("c")
    sid = lax.axis_index("s")
    wid = sid * NUM_CORES + cid
    row0 = wid * ROWS_PER_W
    b = row0 // N
    n_off = row0 % N

    ones = jnp.ones((LANES,), jnp.int32)
    zeros = jnp.zeros((LANES,), jnp.int32)
    bvec = zeros + b

    @plsc.parallel_loop(0, CHUNK // LANES, unroll=8)
    def fill_b(i):
        b_v[pl.ds(i * LANES, LANES)] = bvec

    b_descs = [
        pltpu.async_copy(b_v, idx_hbm.at[0, pl.ds(row0 + k * CHUNK, CHUNK)],
                         sem_b)
        for k in range(N_CHUNKS)
    ]

    def issue_in(k):
        s = k % 2
        return [
            pltpu.async_copy(
                in_hbm.at[c, b, pl.ds(n_off + k * CHUNK, CHUNK)],
                ins[s][c], sem_in[s])
            for c in range(4)
        ]

    def issue_feat(k):
        s = k % 2
        return [
            pltpu.async_copy(
                ins[s][c], feat_hbm.at[c, pl.ds(row0 + k * CHUNK, CHUNK)],
                sem_feat[s])
            for c in range(4)
        ]

    def issue_idx(k):
        s = k % 2
        return [
            pltpu.async_copy(
                qs[s][c], idx_hbm.at[3 - c, pl.ds(row0 + k * CHUNK, CHUNK)],
                sem_idx[s])
            for c in range(3)
        ]

    def compute(k):
        s = k % 2

        @plsc.parallel_loop(0, CHUNK // LANES, unroll=4)
        def quant(i):
            sl = pl.ds(i * LANES, LANES)
            for c in range(3):
                v = (ins[s][c][sl] - _MINS[c]) * GRID_INV
                t = lax.convert_element_type(v, jnp.int32)
                tf = lax.convert_element_type(t, jnp.float32)
                qs[s][c][sl] = t + lax.select(v > tf, ones, zeros)

    in_d = {0: issue_in(0)}
    out_d = {}
    for k in range(N_CHUNKS):
        if k + 1 < N_CHUNKS:
            if k >= 1:
                for d in out_d.pop(k - 1):
                    d.wait()
            in_d[k + 1] = issue_in(k + 1)
        for d in in_d.pop(k):
            d.wait()
        feat = issue_feat(k)
        compute(k)
        out_d[k] = feat + issue_idx(k)
    for key in sorted(out_d):
        for d in out_d[key]:
            d.wait()
    for d in b_descs:
        d.wait()


@jax.jit
def kernel(rdr_sparse_cube):
    planar = jnp.transpose(rdr_sparse_cube, (2, 0, 1))
    mesh = plsc.VectorSubcoreMesh(
        core_axis_name="c", subcore_axis_name="s",
        num_cores=NUM_CORES, num_subcores=NUM_SUBCORES)
    feat_t, idx_t = pl.kernel(
        _sc_body,
        out_type=(
            jax.ShapeDtypeStruct((OUT_C, ROWS), jnp.float32),
            jax.ShapeDtypeStruct((OUT_C, ROWS), jnp.int32),
        ),
        mesh=mesh,
        scratch_types=(
            [pltpu.VMEM((CHUNK,), jnp.float32) for _ in range(8)]
            + [pltpu.VMEM((CHUNK,), jnp.int32) for _ in range(6)]
            + [pltpu.VMEM((CHUNK,), jnp.int32)]
            + [pltpu.SemaphoreType.DMA] * 7
        ),
    )(planar)
    return feat_t.T, idx_t.T


# speedup vs baseline: 1.1242x; 1.0649x over previous
import jax
import jax.numpy as jnp
from jax import lax
from jax.experimental import pallas as pl
from jax.experimental.pallas import tpu as pltpu
from jax.experimental.pallas import tpu_sc as plsc

X_MIN, Y_MIN, Z_MIN = 0.0, -50.0, -2.0
GRID_INV = 2.5

B, N, C = 8, 131072, 5
ROWS = B * N
OUT_C = 4

NUM_CORES = 2
NUM_SUBCORES = 16
NW = NUM_CORES * NUM_SUBCORES
ROWS_PER_W = ROWS // NW
CHUNK = 8192
N_CHUNKS = ROWS_PER_W // CHUNK
LANES = 16

_MINS = (X_MIN, Y_MIN, Z_MIN)


def _sc_body(in_hbm, feat_hbm, idx_hbm, *scratch):
    ins = (scratch[0:4], scratch[4:8])
    qs = (scratch[8:11], scratch[11:14])
    b_v = scratch[14]
    sem_in = scratch[15:17]
    sem_feat = scratch[17:19]
    sem_idx = scratch[19:21]
    sem_b = scratch[21]

    cid = lax.axis_index("c")
    sid = lax.axis_index("s")
    wid = sid * NUM_CORES + cid
    row0 = wid * ROWS_PER_W
    b = row0 // N
    n_off = row0 % N

    ones = jnp.ones((LANES,), jnp.int32)
    zeros = jnp.zeros((LANES,), jnp.int32)
    bvec = zeros + b

    @plsc.parallel_loop(0, CHUNK // LANES, unroll=8)
    def fill_b(i):
        b_v[pl.ds(i * LANES, LANES)] = bvec

    b_descs = [
        pltpu.async_copy(b_v, idx_hbm.at[0, pl.ds(row0 + k * CHUNK, CHUNK)],
                         sem_b)
        for k in range(N_CHUNKS)
    ]

    def issue_in(k):
        s = k % 2
        return [
            pltpu.async_copy(
                in_hbm.at[c, b, pl.ds(n_off + k * CHUNK, CHUNK)],
                ins[s][c], sem_in[s])
            for c in range(4)
        ]

    def feat_copy(k, c):
        s = k % 2
        return pltpu.async_copy(
            ins[s][c], feat_hbm.at[c, pl.ds(row0 + k * CHUNK, CHUNK)],
            sem_feat[s])

    def idx_copy(k, c):
        s = k % 2
        return pltpu.async_copy(
            qs[s][c], idx_hbm.at[3 - c, pl.ds(row0 + k * CHUNK, CHUNK)],
            sem_idx[s])

    def quant_channel(k, c):
        s = k % 2

        @plsc.parallel_loop(0, CHUNK // LANES, unroll=4)
        def quant(i):
            sl = pl.ds(i * LANES, LANES)
            v = (ins[s][c][sl] - _MINS[c]) * GRID_INV
            t = lax.convert_element_type(v, jnp.int32)
            tf = lax.convert_element_type(t, jnp.float32)
            qs[s][c][sl] = t + lax.select(v > tf, ones, zeros)

    in_d = {0: issue_in(0)}
    out_d = {}
    for k in range(N_CHUNKS):
        if k + 1 < N_CHUNKS:
            if k >= 1:
                for d in out_d.pop(k - 1):
                    d.wait()
            in_d[k + 1] = issue_in(k + 1)
        descs = []
        in_descs = in_d.pop(k)
        for c in range(3):
            in_descs[c].wait()
            descs.append(feat_copy(k, c))
            quant_channel(k, c)
            descs.append(idx_copy(k, c))
        in_descs[3].wait()
        descs.append(feat_copy(k, 3))
        out_d[k] = descs
    for key in sorted(out_d):
        for d in out_d[key]:
            d.wait()
    for d in b_descs:
        d.wait()


@jax.jit
def kernel(rdr_sparse_cube):
    planar = jnp.transpose(rdr_sparse_cube, (2, 0, 1))
    mesh = plsc.VectorSubcoreMesh(
        core_axis_name="c", subcore_axis_name="s",
        num_cores=NUM_CORES, num_subcores=NUM_SUBCORES)
    feat_t, idx_t = pl.kernel(
        _sc_body,
        out_type=(
            jax.ShapeDtypeStruct((OUT_C, ROWS), jnp.float32),
            jax.ShapeDtypeStruct((OUT_C, ROWS), jnp.int32),
        ),
        mesh=mesh,
        scratch_types=(
            [pltpu.VMEM((CHUNK,), jnp.float32) for _ in range(8)]
            + [pltpu.VMEM((CHUNK,), jnp.int32) for _ in range(6)]
            + [pltpu.VMEM((CHUNK,), jnp.int32)]
            + [pltpu.SemaphoreType.DMA] * 7
        ),
    )(planar)
    return feat_t.T, idx_t.T


# speedup vs baseline: 1.1544x; 1.0269x over previous
import jax
import jax.numpy as jnp
from jax import lax
from jax.experimental import pallas as pl
from jax.experimental.pallas import tpu as pltpu
from jax.experimental.pallas import tpu_sc as plsc

X_MIN, Y_MIN, Z_MIN = 0.0, -50.0, -2.0
GRID_INV = 2.5

B, N, C = 8, 131072, 5
ROWS = B * N
OUT_C = 4

NUM_CORES = 2
NUM_SUBCORES = 16
NW = NUM_CORES * NUM_SUBCORES
ROWS_PER_W = ROWS // NW
CHUNK = 8192
N_CHUNKS = ROWS_PER_W // CHUNK
LANES = 16

_MINS = (X_MIN, Y_MIN, Z_MIN)


def _sc_body(in_hbm, feat_hbm, idx_hbm, *scratch):
    ins = (scratch[0:4], scratch[4:8])
    qs = (scratch[8:11], scratch[11:14])
    b_v = scratch[14]
    sem_in = scratch[15:17]
    sem_feat = scratch[17:19]
    sem_idx = scratch[19:21]
    sem_b = scratch[21]

    cid = lax.axis_index("c")
    sid = lax.axis_index("s")
    wid = sid * NUM_CORES + cid
    row0 = wid * ROWS_PER_W
    b = row0 // N
    n_off = row0 % N

    ones = jnp.ones((LANES,), jnp.int32)
    zeros = jnp.zeros((LANES,), jnp.int32)
    bvec = zeros + b

    def issue_in(k):
        s = k % 2
        return [
            pltpu.async_copy(
                in_hbm.at[c, b, pl.ds(n_off + k * CHUNK, CHUNK)],
                ins[s][c], sem_in[s])
            for c in range(4)
        ]

    first_in = issue_in(0)

    @plsc.parallel_loop(0, CHUNK // LANES, unroll=8)
    def fill_b(i):
        b_v[pl.ds(i * LANES, LANES)] = bvec

    b_descs = [
        pltpu.async_copy(b_v, idx_hbm.at[0, pl.ds(row0 + k * CHUNK, CHUNK)],
                         sem_b)
        for k in range(N_CHUNKS)
    ]

    def feat_copy(k, c):
        s = k % 2
        return pltpu.async_copy(
            ins[s][c], feat_hbm.at[c, pl.ds(row0 + k * CHUNK, CHUNK)],
            sem_feat[s])

    def idx_copy(k, c):
        s = k % 2
        return pltpu.async_copy(
            qs[s][c], idx_hbm.at[3 - c, pl.ds(row0 + k * CHUNK, CHUNK)],
            sem_idx[s])

    def quant_channel(k, c):
        s = k % 2

        @plsc.parallel_loop(0, CHUNK // LANES, unroll=4)
        def quant(i):
            sl = pl.ds(i * LANES, LANES)
            v = (ins[s][c][sl] - _MINS[c]) * GRID_INV
            t = lax.convert_element_type(v, jnp.int32)
            tf = lax.convert_element_type(t, jnp.float32)
            qs[s][c][sl] = t + lax.select(v > tf, ones, zeros)

    in_d = {0: first_in}
    out_d = {}
    for k in range(N_CHUNKS):
        if k + 1 < N_CHUNKS:
            if k >= 1:
                for d in out_d.pop(k - 1):
                    d.wait()
            in_d[k + 1] = issue_in(k + 1)
        descs = []
        in_descs = in_d.pop(k)
        for c in range(3):
            in_descs[c].wait()
            descs.append(feat_copy(k, c))
            quant_channel(k, c)
            descs.append(idx_copy(k, c))
        in_descs[3].wait()
        descs.append(feat_copy(k, 3))
        out_d[k] = descs
    for key in sorted(out_d):
        for d in out_d[key]:
            d.wait()
    for d in b_descs:
        d.wait()


@jax.jit
def kernel(rdr_sparse_cube):
    planar = jnp.transpose(rdr_sparse_cube, (2, 0, 1))
    mesh = plsc.VectorSubcoreMesh(
        core_axis_name="c", subcore_axis_name="s",
        num_cores=NUM_CORES, num_subcores=NUM_SUBCORES)
    feat_t, idx_t = pl.kernel(
        _sc_body,
        out_type=(
            jax.ShapeDtypeStruct((OUT_C, ROWS), jnp.float32),
            jax.ShapeDtypeStruct((OUT_C, ROWS), jnp.int32),
        ),
        mesh=mesh,
        scratch_types=(
            [pltpu.VMEM((CHUNK,), jnp.float32) for _ in range(8)]
            + [pltpu.VMEM((CHUNK,), jnp.int32) for _ in range(6)]
            + [pltpu.VMEM((CHUNK,), jnp.int32)]
            + [pltpu.SemaphoreType.DMA] * 7
        ),
    )(planar)
    return feat_t.T, idx_t.T
